# Initial kernel scaffold; baseline (speedup 1.0000x reference)
import jax
import jax.numpy as jnp
from jax.experimental import pallas as pl


def kernel(action, action_low, action_high, freqs):
    raise NotImplementedError("write your pallas kernel here")


# speedup vs baseline: 1.0209x; 1.0209x over previous
import jax
import jax.numpy as jnp
from jax.experimental import pallas as pl
from jax.experimental.pallas import tpu as pltpu

_GROUP = 4
_LANES = 128


def _body(a_ref, low_ref, high_ref, ft_ref, o_ref):
    a = a_ref[...]
    low = low_ref[...]
    high = high_ref[...]
    scale = 2.0 / (high - low)
    x = (a - low) * scale + 1.0
    x = jnp.clip(x, -1.0, 1.0)

    lane = jax.lax.broadcasted_iota(jnp.int32, (1, 1, _LANES), 2)
    j = lane // 32
    rep = x[:, :, 0:1] * (j == 0).astype(jnp.float32)
    for k in range(1, _GROUP):
        rep = rep + x[:, :, k:k + 1] * (j == k).astype(jnp.float32)

    arg = rep * ft_ref[...]
    o_ref[:, 0] = jnp.sin(arg)
    o_ref[:, 1] = jnp.cos(arg)


def kernel(action, action_low, action_high, freqs):
    B, A = action.shape
    F = freqs.shape[0]
    G = A // _GROUP
    BB = 512

    a3 = action.reshape(B, G, _GROUP)
    low3 = action_low.reshape(1, G, _GROUP)
    high3 = action_high.reshape(1, G, _GROUP)
    ft = jnp.tile(freqs, _GROUP).reshape(1, _LANES)

    out = pl.pallas_call(
        _body,
        grid=(B // BB,),
        in_specs=[
            pl.BlockSpec((BB, G, _GROUP), lambda i: (i, 0, 0)),
            pl.BlockSpec((1, G, _GROUP), lambda i: (0, 0, 0)),
            pl.BlockSpec((1, G, _GROUP), lambda i: (0, 0, 0)),
            pl.BlockSpec((1, _LANES), lambda i: (0, 0)),
        ],
        out_specs=pl.BlockSpec((BB, 2, G, _LANES), lambda i: (i, 0, 0, 0)),
        out_shape=jax.ShapeDtypeStruct((B, 2, G, _LANES), jnp.float32),
        compiler_params=pltpu.CompilerParams(
            dimension_semantics=("parallel",),
        ),
    )(a3, low3, high3, ft)

    return out.reshape(B, 2 * G * _LANES)


# speedup vs baseline: 1.5321x; 1.5007x over previous
import jax
import jax.numpy as jnp
from jax.experimental import pallas as pl
from jax.experimental.pallas import tpu as pltpu

_L = 8
_NCHAIN = 32 // _L


def _body(ap_ref, low_ref, high_ref, fa_ref, e_ref, kmod_ref, o_ref):
    xa = ap_ref[...]
    low = low_ref[...]
    high = high_ref[...]
    scale = 2.0 / (high - low)
    x = (xa - low) * scale + 1.0
    x = jnp.clip(x, -1.0, 1.0)

    base = x * fa_ref[...]
    s0 = jnp.sin(base)
    c0 = jnp.cos(base)

    s4, c4 = s0, c0
    for _ in range(4):
        s4, c4 = 2.0 * s4 * c4, 2.0 * c4 * c4 - 1.0

    E2 = e_ref[...]

    def expand(z):
        hi = z.astype(jnp.bfloat16)
        lo = (z - hi.astype(jnp.float32)).astype(jnp.bfloat16)
        zz = jnp.concatenate([hi, lo], axis=1)
        return jax.lax.dot_general(zz, E2, (((1,), (0,)), ((), ())),
                                   preferred_element_type=jnp.float32)

    S0 = expand(s0)
    C0 = expand(c0)
    S4 = expand(s4)
    C4 = expand(c4)

    kmod = kmod_ref[...]
    k4 = jnp.where(kmod >= 4, kmod - 4, kmod)
    S = jnp.where(kmod >= 4, S4, S0)
    C = jnp.where(kmod >= 4, C4, C0)
    for j in range(1, 4):
        sd = 2.0 * S * C
        cd = 2.0 * C * C - 1.0
        act = k4 >= j
        S = jnp.where(act, sd, S)
        C = jnp.where(act, cd, C)

    o_ref[:, 0] = S
    o_ref[:, 1] = C


def kernel(action, action_low, action_high, freqs):
    B, A = action.shape
    F = freqs.shape[0]
    P = A * _NCHAIN
    BB = 512

    ap = jnp.repeat(action, _NCHAIN, axis=1)
    lowp = jnp.repeat(action_low, _NCHAIN).reshape(1, P)
    highp = jnp.repeat(action_high, _NCHAIN).reshape(1, P)
    fa = jnp.tile(freqs[::_L], (A,)).reshape(1, P)

    p_idx = jnp.arange(P)[:, None]
    l_idx = jnp.arange(A * F)[None, :]
    E1 = (l_idx // _L == p_idx).astype(jnp.bfloat16)
    E = jnp.concatenate([E1, E1], axis=0)
    kmod = (jnp.arange(A * F, dtype=jnp.int32) % _L).reshape(1, A * F)

    out = pl.pallas_call(
        _body,
        grid=(B // BB,),
        in_specs=[
            pl.BlockSpec((BB, P), lambda i: (i, 0)),
            pl.BlockSpec((1, P), lambda i: (0, 0)),
            pl.BlockSpec((1, P), lambda i: (0, 0)),
            pl.BlockSpec((1, P), lambda i: (0, 0)),
            pl.BlockSpec((2 * P, A * F), lambda i: (0, 0)),
            pl.BlockSpec((1, A * F), lambda i: (0, 0)),
        ],
        out_specs=pl.BlockSpec((BB, 2, A * F), lambda i: (i, 0, 0)),
        out_shape=jax.ShapeDtypeStruct((B, 2, A * F), jnp.float32),
        compiler_params=pltpu.CompilerParams(
            dimension_semantics=("parallel",),
        ),
    )(ap, lowp, highp, fa, E, kmod)

    return out.reshape(B, 2 * A * F)


# speedup vs baseline: 2.8991x; 1.8922x over previous
import jax
import jax.numpy as jnp
from jax.experimental import pallas as pl
from jax.experimental.pallas import tpu as pltpu

_L = 8
_NCHAIN = 32 // _L


def _split3(z):
    hi = z.astype(jnp.bfloat16)
    r1 = z - hi.astype(jnp.float32)
    mid = r1.astype(jnp.bfloat16)
    lo = (r1 - mid.astype(jnp.float32)).astype(jnp.bfloat16)
    return hi, mid, lo


def _split2(z):
    hi = z.astype(jnp.bfloat16)
    lo = (z - hi.astype(jnp.float32)).astype(jnp.bfloat16)
    return hi, lo


def _dot(a, b):
    return jax.lax.dot_general(a, b, (((1,), (0,)), ((), ())),
                               preferred_element_type=jnp.float32)


def _body(a_ref, low_ref, high_ref, fa_ref, r_ref, e_ref, kmod_ref, o_ref):
    a = a_ref[...]
    low = low_ref[...]
    high = high_ref[...]
    scale = 2.0 / (high - low)
    x = (a - low) * scale + 1.0
    x = jnp.clip(x, -1.0, 1.0)

    xh, xm, xl = _split3(x)
    xr = _dot(jnp.concatenate([xh, xm, xl], axis=1), r_ref[...])

    base = xr * fa_ref[...]
    s0 = jnp.sin(base)
    c0 = jnp.cos(base)

    s4, c4 = s0, c0
    for _ in range(4):
        s4, c4 = 2.0 * s4 * c4, 2.0 * c4 * c4 - 1.0

    E2 = e_ref[...]

    def expand(z):
        hi, lo = _split2(z)
        return _dot(jnp.concatenate([hi, lo], axis=1), E2)

    S0 = expand(s0)
    C0 = expand(c0)
    S4 = expand(s4)
    C4 = expand(c4)

    kmod = kmod_ref[...]
    k4 = jnp.where(kmod >= 4, kmod - 4, kmod)
    S = jnp.where(kmod >= 4, S4, S0)
    C = jnp.where(kmod >= 4, C4, C0)
    for j in range(1, 4):
        sd = 2.0 * S * C
        cd = 2.0 * C * C - 1.0
        act = k4 >= j
        S = jnp.where(act, sd, S)
        C = jnp.where(act, cd, C)

    o_ref[:, 0:2048] = S
    o_ref[:, 2048:4096] = C


def kernel(action, action_low, action_high, freqs):
    B, A = action.shape
    F = freqs.shape[0]
    P = A * _NCHAIN
    BB = 512

    low2 = action_low.reshape(1, A)
    high2 = action_high.reshape(1, A)
    fa = jnp.tile(freqs[::_L], (A,)).reshape(1, P)

    a_idx = jnp.arange(A)[:, None]
    p_idx = jnp.arange(P)[None, :]
    R1 = (p_idx // _NCHAIN == a_idx).astype(jnp.bfloat16)
    R = jnp.concatenate([R1, R1, R1], axis=0)

    pp_idx = jnp.arange(P)[:, None]
    l_idx = jnp.arange(A * F)[None, :]
    E1 = (l_idx // _L == pp_idx).astype(jnp.bfloat16)
    E = jnp.concatenate([E1, E1], axis=0)
    kmod = (jnp.arange(A * F, dtype=jnp.int32) % _L).reshape(1, A * F)

    out = pl.pallas_call(
        _body,
        grid=(B // BB,),
        in_specs=[
            pl.BlockSpec((BB, A), lambda i: (i, 0)),
            pl.BlockSpec((1, A), lambda i: (0, 0)),
            pl.BlockSpec((1, A), lambda i: (0, 0)),
            pl.BlockSpec((1, P), lambda i: (0, 0)),
            pl.BlockSpec((3 * A, P), lambda i: (0, 0)),
            pl.BlockSpec((2 * P, A * F), lambda i: (0, 0)),
            pl.BlockSpec((1, A * F), lambda i: (0, 0)),
        ],
        out_specs=pl.BlockSpec((BB, 2 * A * F), lambda i: (i, 0)),
        out_shape=jax.ShapeDtypeStruct((B, 2 * A * F), jnp.float32),
        compiler_params=pltpu.CompilerParams(
            dimension_semantics=("parallel",),
        ),
    )(action, low2, high2, fa, R, E, kmod)

    return out
